# Initial kernel scaffold; baseline (speedup 1.0000x reference)
import jax
import jax.numpy as jnp
from jax.experimental import pallas as pl


def kernel(indices, prototypes):
    raise NotImplementedError("write your pallas kernel here")


# speedup vs baseline: 4.8617x; 4.8617x over previous
import functools

import jax
import jax.numpy as jnp
from jax import lax
from jax.experimental import pallas as pl
from jax.experimental.pallas import tpu as pltpu
from jax.experimental.pallas import tpu_sc as plsc

_D = 32
_NC = 2
_NS = 16
_NW = _NC * _NS
_CHUNK = 1280


@functools.lru_cache(maxsize=None)
def _gather_call(B):
    assert B % (_NW * _CHUNK) == 0
    b_per_w = B // _NW
    n_chunks = b_per_w // _CHUNK
    mesh = plsc.VectorSubcoreMesh(core_axis_name="c", subcore_axis_name="s")

    @functools.partial(
        pl.kernel,
        mesh=mesh,
        out_type=jax.ShapeDtypeStruct((B, _D), jnp.float32),
        scratch_types=[
            pltpu.VMEM((_CHUNK,), jnp.int32),
            pltpu.VMEM((_CHUNK, _D), jnp.float32),
            pltpu.SemaphoreType.DMA,
        ],
        compiler_params=pltpu.CompilerParams(use_tc_tiling_on_sc=False),
    )
    def k(idx_hbm, table_hbm, out_hbm, idx_v, rows_v, sem):
        wid = lax.axis_index("s") * _NC + lax.axis_index("c")
        base = wid * b_per_w

        def body(g, carry):
            start = base + g * _CHUNK
            pltpu.sync_copy(idx_hbm.at[pl.ds(start, _CHUNK)], idx_v)
            pltpu.async_copy(table_hbm.at[idx_v], rows_v, sem).wait()
            pltpu.sync_copy(rows_v, out_hbm.at[pl.ds(start, _CHUNK)])
            return carry

        lax.fori_loop(0, n_chunks, body, 0)

    return k


def kernel(indices, prototypes):
    bsz, hist = indices.shape
    flat_idx = indices.reshape(bsz * hist).astype(jnp.int32)
    out = _gather_call(bsz * hist)(flat_idx, prototypes)
    return out.reshape(bsz, hist, _D)


# speedup vs baseline: 5.0493x; 1.0386x over previous
import functools

import jax
import jax.numpy as jnp
from jax import lax
from jax.experimental import pallas as pl
from jax.experimental.pallas import tpu as pltpu
from jax.experimental.pallas import tpu_sc as plsc

_D = 32
_NC = 2
_NS = 16
_NW = _NC * _NS
_CHUNK = 800
_NB = 4


@functools.lru_cache(maxsize=None)
def _gather_call(B):
    assert B % (_NW * _CHUNK) == 0
    b_per_w = B // _NW
    n = b_per_w // _CHUNK
    assert n % _NB == 0 and n >= 3 * _NB
    mesh = plsc.VectorSubcoreMesh(core_axis_name="c", subcore_axis_name="s")

    scratch = (
        [pltpu.VMEM((_CHUNK,), jnp.int32) for _ in range(_NB)]
        + [pltpu.VMEM((_CHUNK, _D), jnp.float32) for _ in range(_NB)]
        + [pltpu.SemaphoreType.DMA for _ in range(3 * _NB)]
    )

    @functools.partial(
        pl.kernel,
        mesh=mesh,
        out_type=jax.ShapeDtypeStruct((B, _D), jnp.float32),
        scratch_types=scratch,
        compiler_params=pltpu.CompilerParams(use_tc_tiling_on_sc=False),
    )
    def k(idx_hbm, table_hbm, out_hbm, *s):
        idxv = s[0:_NB]
        rows = s[_NB:2 * _NB]
        sem_i = s[2 * _NB:3 * _NB]
        sem_g = s[3 * _NB:4 * _NB]
        sem_o = s[4 * _NB:5 * _NB]

        wid = lax.axis_index("s") * _NC + lax.axis_index("c")
        base = wid * b_per_w

        def i_start(c, b):
            pltpu.make_async_copy(
                idx_hbm.at[pl.ds(base + c * _CHUNK, _CHUNK)], idxv[b], sem_i[b]
            ).start()

        def i_wait(b):
            pltpu.make_async_copy(
                idx_hbm.at[pl.ds(base, _CHUNK)], idxv[b], sem_i[b]
            ).wait()

        def g_start(b):
            pltpu.make_async_copy(table_hbm.at[idxv[b]], rows[b], sem_g[b]).start()

        def g_wait(b):
            pltpu.make_async_copy(table_hbm.at[idxv[b]], rows[b], sem_g[b]).wait()

        def w_start(c, b):
            pltpu.make_async_copy(
                rows[b], out_hbm.at[pl.ds(base + c * _CHUNK, _CHUNK)], sem_o[b]
            ).start()

        def w_wait(b):
            pltpu.make_async_copy(
                rows[b], out_hbm.at[pl.ds(base, _CHUNK)], sem_o[b]
            ).wait()

        def step(c, b, do_i, do_wwait, do_g):
            b1 = (b + 1) % _NB
            if do_wwait:
                w_wait(b1)
            if do_g:
                i_wait(b1)
                g_start(b1)
            g_wait(b)
            w_start(c, b)
            if do_i:
                i_start(c + _NB, b)

        for b in range(_NB):
            i_start(b, b)
        i_wait(0)
        g_start(0)

        for c in range(_NB):
            step(c, c % _NB, do_i=True, do_wwait=(c >= 3), do_g=True)

        @pl.loop(_NB, n - _NB, step=_NB)
        def _steady(sbase):
            for b in range(_NB):
                step(sbase + b, b, do_i=True, do_wwait=True, do_g=True)

        for c in range(n - _NB, n):
            step(c, c % _NB, do_i=False, do_wwait=True, do_g=(c + 1 < n))

        for c in range(n - 3, n):
            w_wait(c % _NB)

    return k


def kernel(indices, prototypes):
    bsz, hist = indices.shape
    flat_idx = indices.reshape(bsz * hist).astype(jnp.int32)
    out = _gather_call(bsz * hist)(flat_idx, prototypes)
    return out.reshape(bsz, hist, _D)
